# baseline (device time: 405037 ns/iter reference)
import jax
import jax.numpy as jnp
from jax import lax
from jax.experimental import pallas as pl
from jax.experimental.pallas import tpu as pltpu

N_DEV = 32
N_FLOW = 8
N_SLOT = 3


def kernel(x, w_mat, scale_x, scale_w):
    m, _ = x.shape
    _, n = w_mat.shape
    chunk = m // N_DEV
    sub = chunk // N_FLOW

    def body(x_ref, w_ref, sx_ref, sw_ref, out_ref,
             comm, stage, send_sems, recv_sems, credits):
        my = lax.axis_index("i")
        left = lax.rem(my - 1 + N_DEV, N_DEV)
        right = lax.rem(my + 1, N_DEV)

        barrier_sem = pltpu.get_barrier_semaphore()
        for nbr in (left, right):
            pl.semaphore_signal(barrier_sem, inc=1, device_id=(nbr,),
                                device_id_type=pl.DeviceIdType.MESH)
        pl.semaphore_wait(barrier_sem, 2)

        scale = sx_ref[0] * sw_ref[0]
        acc = lax.dot_general(
            (x_ref[...] * scale).astype(jnp.bfloat16),
            w_ref[...].astype(jnp.bfloat16),
            dimension_numbers=(((1,), (0,)), ((), ())),
            preferred_element_type=jnp.float32,
        )
        out_ref[...] = acc

        n_steps = 2 * (N_DEV - 1)

        def indices(u):
            if u < N_DEV - 1:
                s = u
                return (lax.rem(my - s + N_DEV, N_DEV),
                        lax.rem(my - s - 1 + N_DEV, N_DEV))
            t = u - (N_DEV - 1)
            return (lax.rem(my + 1 - t + 2 * N_DEV, N_DEV),
                    lax.rem(my - t + 2 * N_DEV, N_DEV))

        def rows(idx, h):
            return pl.ds(idx * chunk + h * sub, sub)

        def launch(u, h):
            send_idx, _ = indices(u)
            if u >= N_SLOT:
                pl.semaphore_wait(credits.at[h], 1)
            stage[h, u % 2] = out_ref[rows(send_idx, h), :].astype(
                jnp.bfloat16)
            rdma = pltpu.make_async_remote_copy(
                src_ref=stage.at[h, u % 2],
                dst_ref=comm.at[h, u % N_SLOT],
                send_sem=send_sems.at[h, u % 2],
                recv_sem=recv_sems.at[h, u % N_SLOT],
                device_id=(right,),
                device_id_type=pl.DeviceIdType.MESH,
            )
            rdma.start()
            return rdma

        def finish(u, h, rdma):
            _, recv_idx = indices(u)
            rdma.wait()
            inbound = comm[h, u % N_SLOT].astype(jnp.float32)
            if u < N_DEV - 1:
                out_ref[rows(recv_idx, h), :] = (
                    out_ref[rows(recv_idx, h), :] + inbound
                )
            else:
                out_ref[rows(recv_idx, h), :] = inbound
            if u < n_steps - N_SLOT:
                pl.semaphore_signal(credits.at[h], inc=1, device_id=(left,),
                                    device_id_type=pl.DeviceIdType.MESH)

        pending = [launch(0, h) for h in range(N_FLOW)]
        for u in range(1, n_steps):
            for h in range(N_FLOW):
                finish(u - 1, h, pending[h])
                pending[h] = launch(u, h)
        for h in range(N_FLOW):
            finish(n_steps - 1, h, pending[h])

    return pl.pallas_call(
        body,
        out_shape=jax.ShapeDtypeStruct((m, n), jnp.float32),
        in_specs=[
            pl.BlockSpec(memory_space=pltpu.VMEM),
            pl.BlockSpec(memory_space=pltpu.VMEM),
            pl.BlockSpec(memory_space=pltpu.SMEM),
            pl.BlockSpec(memory_space=pltpu.SMEM),
        ],
        out_specs=pl.BlockSpec(memory_space=pltpu.VMEM),
        scratch_shapes=[
            pltpu.VMEM((N_FLOW, N_SLOT, sub, n), jnp.bfloat16),
            pltpu.VMEM((N_FLOW, 2, sub, n), jnp.bfloat16),
            pltpu.SemaphoreType.DMA((N_FLOW, 2)),
            pltpu.SemaphoreType.DMA((N_FLOW, N_SLOT)),
            pltpu.SemaphoreType.REGULAR((N_FLOW,)),
        ],
        compiler_params=pltpu.CompilerParams(
            collective_id=0,
            vmem_limit_bytes=100 * 1024 * 1024,
        ),
    )(x, w_mat, scale_x, scale_w)


# device time: 402118 ns/iter; 1.0073x vs baseline; 1.0073x over previous
import jax
import jax.numpy as jnp
from jax import lax
from jax.experimental import pallas as pl
from jax.experimental.pallas import tpu as pltpu

N_DEV = 32
N_FLOW = 4
N_SLOT = 3


def kernel(x, w_mat, scale_x, scale_w):
    m, _ = x.shape
    _, n = w_mat.shape
    chunk = m // N_DEV
    sub = chunk // N_FLOW

    def body(x_ref, w_ref, sx_ref, sw_ref, out_ref,
             part_ref, comm, send_sems, recv_sems, credits):
        my = lax.axis_index("i")
        left = lax.rem(my - 1 + N_DEV, N_DEV)
        right = lax.rem(my + 1, N_DEV)

        barrier_sem = pltpu.get_barrier_semaphore()
        for nbr in (left, right):
            pl.semaphore_signal(barrier_sem, inc=1, device_id=(nbr,),
                                device_id_type=pl.DeviceIdType.MESH)
        pl.semaphore_wait(barrier_sem, 2)

        scale = sx_ref[0] * sw_ref[0]
        acc = lax.dot_general(
            (x_ref[...] * scale).astype(jnp.bfloat16),
            w_ref[...].astype(jnp.bfloat16),
            dimension_numbers=(((1,), (0,)), ((), ())),
            preferred_element_type=jnp.float32,
        )
        part_ref[...] = acc.astype(jnp.bfloat16)

        n_steps = 2 * (N_DEV - 1)

        def indices(u):
            if u < N_DEV - 1:
                s = u
                return (lax.rem(my - s + N_DEV, N_DEV),
                        lax.rem(my - s - 1 + N_DEV, N_DEV))
            t = u - (N_DEV - 1)
            return (lax.rem(my + 1 - t + 2 * N_DEV, N_DEV),
                    lax.rem(my - t + 2 * N_DEV, N_DEV))

        def rows(idx, h):
            return pl.ds(idx * chunk + h * sub, sub)

        def launch(u, h):
            send_idx, _ = indices(u)
            if u >= N_SLOT:
                pl.semaphore_wait(credits.at[h], 1)
            rdma = pltpu.make_async_remote_copy(
                src_ref=part_ref.at[rows(send_idx, h), :],
                dst_ref=comm.at[h, u % N_SLOT],
                send_sem=send_sems.at[h, u % 2],
                recv_sem=recv_sems.at[h, u % N_SLOT],
                device_id=(right,),
                device_id_type=pl.DeviceIdType.MESH,
            )
            rdma.start()
            return rdma

        def finish(u, h, rdma):
            _, recv_idx = indices(u)
            rdma.wait()
            if u < N_DEV - 1:
                part_ref[rows(recv_idx, h), :] = (
                    part_ref[rows(recv_idx, h), :] + comm[h, u % N_SLOT]
                )
            else:
                part_ref[rows(recv_idx, h), :] = comm[h, u % N_SLOT]
                out_ref[rows(recv_idx, h), :] = comm[h, u % N_SLOT].astype(
                    jnp.float32)
            if u < n_steps - N_SLOT:
                pl.semaphore_signal(credits.at[h], inc=1, device_id=(left,),
                                    device_id_type=pl.DeviceIdType.MESH)

        pending = [launch(0, h) for h in range(N_FLOW)]
        for u in range(1, n_steps):
            for h in range(N_FLOW):
                finish(u - 1, h, pending[h])
                pending[h] = launch(u, h)
        for h in range(N_FLOW):
            finish(n_steps - 1, h, pending[h])

        own = lax.rem(my + 1, N_DEV)
        for h in range(N_FLOW):
            out_ref[rows(own, h), :] = part_ref[rows(own, h), :].astype(
                jnp.float32)

    return pl.pallas_call(
        body,
        out_shape=jax.ShapeDtypeStruct((m, n), jnp.float32),
        in_specs=[
            pl.BlockSpec(memory_space=pltpu.VMEM),
            pl.BlockSpec(memory_space=pltpu.VMEM),
            pl.BlockSpec(memory_space=pltpu.SMEM),
            pl.BlockSpec(memory_space=pltpu.SMEM),
        ],
        out_specs=pl.BlockSpec(memory_space=pltpu.VMEM),
        scratch_shapes=[
            pltpu.VMEM((m, n), jnp.bfloat16),
            pltpu.VMEM((N_FLOW, N_SLOT, sub, n), jnp.bfloat16),
            pltpu.SemaphoreType.DMA((N_FLOW, 2)),
            pltpu.SemaphoreType.DMA((N_FLOW, N_SLOT)),
            pltpu.SemaphoreType.REGULAR((N_FLOW,)),
        ],
        compiler_params=pltpu.CompilerParams(
            collective_id=0,
            vmem_limit_bytes=100 * 1024 * 1024,
        ),
    )(x, w_mat, scale_x, scale_w)


# device time: 400730 ns/iter; 1.0107x vs baseline; 1.0035x over previous
import jax
import jax.numpy as jnp
from jax import lax
from jax.experimental import pallas as pl
from jax.experimental.pallas import tpu as pltpu

N_DEV = 32
N_FLOW = 2


def kernel(x, w_mat, scale_x, scale_w):
    m, _ = x.shape
    _, n = w_mat.shape
    chunk = m // N_DEV
    sub = chunk // N_FLOW

    def body(x_ref, w_ref, sx_ref, sw_ref, out_ref,
             comm, stage, send_sems, recv_sems, credits):
        my = lax.axis_index("i")
        left = lax.rem(my - 1 + N_DEV, N_DEV)
        right = lax.rem(my + 1, N_DEV)

        barrier_sem = pltpu.get_barrier_semaphore()
        for nbr in (left, right):
            pl.semaphore_signal(barrier_sem, inc=1, device_id=(nbr,),
                                device_id_type=pl.DeviceIdType.MESH)
        pl.semaphore_wait(barrier_sem, 2)

        scale = sx_ref[0] * sw_ref[0]
        acc = lax.dot_general(
            x_ref[...] * scale, w_ref[...],
            dimension_numbers=(((1,), (0,)), ((), ())),
            preferred_element_type=jnp.float32,
        )
        out_ref[...] = acc

        n_steps = 2 * (N_DEV - 1)

        def indices(u):
            if u < N_DEV - 1:
                s = u
                return (lax.rem(my - s + N_DEV, N_DEV),
                        lax.rem(my - s - 1 + N_DEV, N_DEV))
            t = u - (N_DEV - 1)
            return (lax.rem(my + 1 - t + 2 * N_DEV, N_DEV),
                    lax.rem(my - t + 2 * N_DEV, N_DEV))

        def rows(idx, h):
            return pl.ds(idx * chunk + h * sub, sub)

        def launch(u, h):
            send_idx, _ = indices(u)
            if u >= 2:
                pl.semaphore_wait(credits.at[h], 1)
            stage[h, u % 2] = out_ref[rows(send_idx, h), :].astype(
                jnp.bfloat16)
            rdma = pltpu.make_async_remote_copy(
                src_ref=stage.at[h, u % 2],
                dst_ref=comm.at[h, u % 2],
                send_sem=send_sems.at[h, u % 2],
                recv_sem=recv_sems.at[h, u % 2],
                device_id=(right,),
                device_id_type=pl.DeviceIdType.MESH,
            )
            rdma.start()
            return rdma

        def finish(u, h, rdma):
            _, recv_idx = indices(u)
            rdma.wait()
            inbound = comm[h, u % 2].astype(jnp.float32)
            if u < N_DEV - 1:
                out_ref[rows(recv_idx, h), :] = (
                    out_ref[rows(recv_idx, h), :] + inbound
                )
            else:
                out_ref[rows(recv_idx, h), :] = inbound
            if u < n_steps - 2:
                pl.semaphore_signal(credits.at[h], inc=1, device_id=(left,),
                                    device_id_type=pl.DeviceIdType.MESH)

        pending = [launch(0, h) for h in range(N_FLOW)]
        for u in range(1, n_steps):
            for h in range(N_FLOW):
                finish(u - 1, h, pending[h])
                pending[h] = launch(u, h)
        for h in range(N_FLOW):
            finish(n_steps - 1, h, pending[h])

    return pl.pallas_call(
        body,
        out_shape=jax.ShapeDtypeStruct((m, n), jnp.float32),
        in_specs=[
            pl.BlockSpec(memory_space=pltpu.VMEM),
            pl.BlockSpec(memory_space=pltpu.VMEM),
            pl.BlockSpec(memory_space=pltpu.SMEM),
            pl.BlockSpec(memory_space=pltpu.SMEM),
        ],
        out_specs=pl.BlockSpec(memory_space=pltpu.VMEM),
        scratch_shapes=[
            pltpu.VMEM((N_FLOW, 2, sub, n), jnp.bfloat16),
            pltpu.VMEM((N_FLOW, 2, sub, n), jnp.bfloat16),
            pltpu.SemaphoreType.DMA((N_FLOW, 2)),
            pltpu.SemaphoreType.DMA((N_FLOW, 2)),
            pltpu.SemaphoreType.REGULAR((N_FLOW,)),
        ],
        compiler_params=pltpu.CompilerParams(
            collective_id=0,
            vmem_limit_bytes=100 * 1024 * 1024,
        ),
    )(x, w_mat, scale_x, scale_w)


# device time: 396346 ns/iter; 1.0219x vs baseline; 1.0111x over previous
import jax
import jax.numpy as jnp
from jax import lax
from jax.experimental import pallas as pl
from jax.experimental.pallas import tpu as pltpu

N_DEV = 32
N_FLOW = 2


def kernel(x, w_mat, scale_x, scale_w):
    m, _ = x.shape
    _, n = w_mat.shape
    chunk = m // N_DEV
    sub = chunk // N_FLOW

    def body(x_ref, w_ref, sx_ref, sw_ref, out_ref,
             comm, stage, send_sems, recv_sems, credits):
        my = lax.axis_index("i")
        left = lax.rem(my - 1 + N_DEV, N_DEV)
        right = lax.rem(my + 1, N_DEV)

        barrier_sem = pltpu.get_barrier_semaphore()
        for nbr in (left, right):
            pl.semaphore_signal(barrier_sem, inc=1, device_id=(nbr,),
                                device_id_type=pl.DeviceIdType.MESH)
        pl.semaphore_wait(barrier_sem, 2)

        scale = sx_ref[0] * sw_ref[0]
        x_bf = (x_ref[...] * scale).astype(jnp.bfloat16)
        w_bf = w_ref[...].astype(jnp.bfloat16)

        n_steps = 2 * (N_DEV - 1)

        def indices(u):
            if u < N_DEV - 1:
                s = u
                return (lax.rem(my - s + N_DEV, N_DEV),
                        lax.rem(my - s - 1 + N_DEV, N_DEV))
            t = u - (N_DEV - 1)
            return (lax.rem(my + 1 - t + 2 * N_DEV, N_DEV),
                    lax.rem(my - t + 2 * N_DEV, N_DEV))

        def rows(idx, h):
            return pl.ds(idx * chunk + h * sub, sub)

        def launch(u, h):
            send_idx, _ = indices(u)
            if u >= 2:
                pl.semaphore_wait(credits.at[h], 1)
            stage[h, u % 2] = out_ref[rows(send_idx, h), :].astype(
                jnp.bfloat16)
            rdma = pltpu.make_async_remote_copy(
                src_ref=stage.at[h, u % 2],
                dst_ref=comm.at[h, u % 2],
                send_sem=send_sems.at[h, u % 2],
                recv_sem=recv_sems.at[h, u % 2],
                device_id=(right,),
                device_id_type=pl.DeviceIdType.MESH,
            )
            rdma.start()
            return rdma

        def finish(u, h, rdma):
            _, recv_idx = indices(u)
            rdma.wait()
            inbound = comm[h, u % 2].astype(jnp.float32)
            if u < N_DEV - 1:
                out_ref[rows(recv_idx, h), :] = (
                    out_ref[rows(recv_idx, h), :] + inbound
                )
            else:
                out_ref[rows(recv_idx, h), :] = inbound
            if u < n_steps - 2:
                pl.semaphore_signal(credits.at[h], inc=1, device_id=(left,),
                                    device_id_type=pl.DeviceIdType.MESH)

        x_my = (x_ref[pl.ds(my * chunk, chunk), :] * scale).astype(
            jnp.bfloat16)
        out_ref[pl.ds(my * chunk, chunk), :] = lax.dot_general(
            x_my, w_bf,
            dimension_numbers=(((1,), (0,)), ((), ())),
            preferred_element_type=jnp.float32,
        )
        pending = [launch(0, h) for h in range(N_FLOW)]
        out_ref[...] = lax.dot_general(
            x_bf, w_bf,
            dimension_numbers=(((1,), (0,)), ((), ())),
            preferred_element_type=jnp.float32,
        )

        for u in range(1, n_steps):
            for h in range(N_FLOW):
                finish(u - 1, h, pending[h])
                pending[h] = launch(u, h)
        for h in range(N_FLOW):
            finish(n_steps - 1, h, pending[h])

    return pl.pallas_call(
        body,
        out_shape=jax.ShapeDtypeStruct((m, n), jnp.float32),
        in_specs=[
            pl.BlockSpec(memory_space=pltpu.VMEM),
            pl.BlockSpec(memory_space=pltpu.VMEM),
            pl.BlockSpec(memory_space=pltpu.SMEM),
            pl.BlockSpec(memory_space=pltpu.SMEM),
        ],
        out_specs=pl.BlockSpec(memory_space=pltpu.VMEM),
        scratch_shapes=[
            pltpu.VMEM((N_FLOW, 2, sub, n), jnp.bfloat16),
            pltpu.VMEM((N_FLOW, 2, sub, n), jnp.bfloat16),
            pltpu.SemaphoreType.DMA((N_FLOW, 2)),
            pltpu.SemaphoreType.DMA((N_FLOW, 2)),
            pltpu.SemaphoreType.REGULAR((N_FLOW,)),
        ],
        compiler_params=pltpu.CompilerParams(
            collective_id=0,
            vmem_limit_bytes=100 * 1024 * 1024,
        ),
    )(x, w_mat, scale_x, scale_w)
